# Initial kernel scaffold; baseline (speedup 1.0000x reference)
import jax
import jax.numpy as jnp
from jax.experimental import pallas as pl


def kernel(E, v, m_logits, cls_W, cls_b, beta, alpha):
    raise NotImplementedError("write your pallas kernel here")


# speedup vs baseline: 3.4130x; 3.4130x over previous
import functools

import jax
import jax.numpy as jnp
from jax.experimental import pallas as pl

_K = 4
_TOP_M = 8
_TEMP = 0.07
_EPS = 1e-06
_NUM_CLASSES = 1000


def _matmul_kernel(e_ref, u_ref, out_ref):
    e = e_ref[0]
    u = u_ref[...]
    out_ref[0] = jax.lax.dot_general(
        u, e, (((1,), (1,)), ((), ())),
        preferred_element_type=jnp.float32,
        precision=jax.lax.Precision.HIGHEST,
    )


def _finish_kernel(c0_ref, u_ref, clsw_ref, clsb_ref, beta_ref, alpha_ref,
                   out_ref, *, B, T, top_m):
    u = u_ref[...]
    gram = jax.lax.dot_general(
        u, u, (((1,), (1,)), ((), ())),
        preferred_element_type=jnp.float32,
        precision=jax.lax.Precision.HIGHEST,
    )
    beta = beta_ref[...]
    alpha = alpha_ref[...]
    clsb = clsb_ref[...]

    iota = jax.lax.broadcasted_iota(jnp.int32, (B, T), 1)
    neg_inf = jnp.float32(-jnp.inf)

    coeffs = []
    logits = jnp.zeros((B, _NUM_CLASSES), dtype=jnp.float32)
    for k in range(_K):
        ck = c0_ref[0, k]
        for j in range(k):
            ck = ck - (beta[0, j] * gram[j, k]) * coeffs[j]
        coeffs.append(ck)

        m = jnp.max(ck, axis=1, keepdims=True)
        e = jnp.exp((ck - m) * (1.0 / _TEMP))
        z = jnp.sum(e, axis=1, keepdims=True)
        s_soft = jnp.sum(e * ck, axis=1, keepdims=True) / z

        cur = ck
        s_top = jnp.zeros((B, 1), dtype=jnp.float32)
        for _ in range(top_m):
            mx = jnp.max(cur, axis=1, keepdims=True)
            s_top = s_top + mx
            hit = jnp.where(cur == mx, iota, T)
            first = jnp.min(hit, axis=1, keepdims=True)
            cur = jnp.where(iota == first, neg_inf, cur)

        c_bk = 0.5 * (s_top + s_soft)

        wu = jax.lax.dot_general(
            u[k:k + 1], clsw_ref[k], (((1,), (1,)), ((), ())),
            preferred_element_type=jnp.float32,
            precision=jax.lax.Precision.HIGHEST,
        )
        logits = logits + alpha[0, k] * (c_bk * wu + clsb[k][None, :])

    out_ref[...] = logits


def kernel(E, v, m_logits, cls_W, cls_b, beta, alpha):
    B, T, D = E.shape
    K = v.shape[0]
    top_m = min(_TOP_M, T)

    mk = jax.nn.sigmoid(m_logits)
    vk = v * mk
    U = vk / (jnp.linalg.norm(vk, axis=1, keepdims=True) + _EPS)

    Tb = 1024
    c0 = pl.pallas_call(
        _matmul_kernel,
        grid=(B, T // Tb),
        in_specs=[
            pl.BlockSpec((1, Tb, D), lambda b, t: (b, t, 0)),
            pl.BlockSpec((K, D), lambda b, t: (0, 0)),
        ],
        out_specs=pl.BlockSpec((1, K, Tb), lambda b, t: (b, 0, t)),
        out_shape=jax.ShapeDtypeStruct((B, K, T), jnp.float32),
    )(E, U)

    c0_kbt = c0.transpose(1, 0, 2)[None]

    finish = functools.partial(_finish_kernel, B=B, T=T, top_m=top_m)
    logits = pl.pallas_call(
        finish,
        out_shape=jax.ShapeDtypeStruct((B, _NUM_CLASSES), jnp.float32),
    )(c0_kbt, U, cls_W, cls_b, beta.reshape(1, K), alpha.reshape(1, K))
    return logits


# speedup vs baseline: 7.2864x; 2.1349x over previous
import functools

import jax
import jax.numpy as jnp
from jax.experimental import pallas as pl

_K = 4
_TOP_M = 8
_TEMP = 0.07
_EPS = 1e-06
_NUM_CLASSES = 1000


def _matmul_kernel(e_ref, u_ref, out_ref):
    e = e_ref[0]
    u = u_ref[...]
    out_ref[0] = jax.lax.dot_general(
        u, e, (((1,), (1,)), ((), ())),
        preferred_element_type=jnp.float32,
    )


def _finish_kernel(c0_ref, u_ref, clsw_ref, clsb_ref, beta_ref, alpha_ref,
                   out_ref, *, B, T, top_m):
    u = u_ref[...]
    gram = jax.lax.dot_general(
        u, u, (((1,), (1,)), ((), ())),
        preferred_element_type=jnp.float32,
        precision=jax.lax.Precision.HIGHEST,
    )
    beta = beta_ref[...]
    alpha = alpha_ref[...]
    clsb = clsb_ref[...]

    iota = jax.lax.broadcasted_iota(jnp.int32, (B, T), 1)
    neg_inf = jnp.float32(-jnp.inf)

    coeffs = []
    logits = jnp.zeros((B, _NUM_CLASSES), dtype=jnp.float32)
    for k in range(_K):
        ck = c0_ref[0, k]
        for j in range(k):
            ck = ck - (beta[0, j] * gram[j, k]) * coeffs[j]
        coeffs.append(ck)

        m = jnp.max(ck, axis=1, keepdims=True)
        e = jnp.exp((ck - m) * (1.0 / _TEMP))
        z = jnp.sum(e, axis=1, keepdims=True)
        s_soft = jnp.sum(e * ck, axis=1, keepdims=True) / z

        cur = ck
        s_top = jnp.zeros((B, 1), dtype=jnp.float32)
        for _ in range(top_m):
            mx = jnp.max(cur, axis=1, keepdims=True)
            s_top = s_top + mx
            hit = jnp.where(cur == mx, iota, T)
            first = jnp.min(hit, axis=1, keepdims=True)
            cur = jnp.where(iota == first, neg_inf, cur)

        c_bk = 0.5 * (s_top + s_soft)

        wu = jax.lax.dot_general(
            u[k:k + 1], clsw_ref[k], (((1,), (1,)), ((), ())),
            preferred_element_type=jnp.float32,
        )
        logits = logits + alpha[0, k] * (c_bk * wu + clsb[k][None, :])

    out_ref[...] = logits


def kernel(E, v, m_logits, cls_W, cls_b, beta, alpha):
    B, T, D = E.shape
    K = v.shape[0]
    top_m = min(_TOP_M, T)

    mk = jax.nn.sigmoid(m_logits)
    vk = v * mk
    U = vk / (jnp.linalg.norm(vk, axis=1, keepdims=True) + _EPS)

    Tb = 1024
    c0 = pl.pallas_call(
        _matmul_kernel,
        grid=(B, T // Tb),
        in_specs=[
            pl.BlockSpec((1, Tb, D), lambda b, t: (b, t, 0)),
            pl.BlockSpec((K, D), lambda b, t: (0, 0)),
        ],
        out_specs=pl.BlockSpec((1, K, Tb), lambda b, t: (b, 0, t)),
        out_shape=jax.ShapeDtypeStruct((B, K, T), jnp.float32),
    )(E, U)

    c0_kbt = c0.transpose(1, 0, 2)[None]

    finish = functools.partial(_finish_kernel, B=B, T=T, top_m=top_m)
    logits = pl.pallas_call(
        finish,
        out_shape=jax.ShapeDtypeStruct((B, _NUM_CLASSES), jnp.float32),
    )(c0_kbt, U, cls_W, cls_b, beta.reshape(1, K), alpha.reshape(1, K))
    return logits


# speedup vs baseline: 7.9284x; 1.0881x over previous
import functools

import jax
import jax.numpy as jnp
from jax.experimental import pallas as pl
from jax.experimental.pallas import tpu as pltpu

_K = 4
_TOP_M = 8
_TEMP = 0.07
_EPS = 1e-06
_NUM_CLASSES = 1000


def _matmul_kernel(e_ref, u_ref, out_ref):
    e = e_ref[0]
    u = u_ref[...]
    out_ref[0] = jax.lax.dot_general(
        u, e, (((1,), (1,)), ((), ())),
        preferred_element_type=jnp.float32,
    )


def _finish_step(kk, c0_ref, u_ref, clsw_ref, clsb_ref, beta_ref, alpha_ref,
                 out_ref, coeff_ref, *, B, T, top_m):
    u = u_ref[...]
    beta = beta_ref[...]
    alpha = alpha_ref[...]

    ck = c0_ref[kk]
    for j in range(kk):
        g_jk = jnp.sum(u[j] * u[kk])
        ck = ck - (beta[0, j] * g_jk) * coeff_ref[j]
    if kk + 1 < _K:
        coeff_ref[kk] = ck

    m = jnp.max(ck, axis=1, keepdims=True)
    e = jnp.exp((ck - m) * (1.0 / _TEMP))
    z = jnp.sum(e, axis=1, keepdims=True)
    s_soft = jnp.sum(e * ck, axis=1, keepdims=True) / z

    iota = jax.lax.broadcasted_iota(jnp.int32, (B, T), 1)
    cur = ck
    s_top = jnp.zeros((B, 1), dtype=jnp.float32)
    for _ in range(top_m):
        mx = jnp.max(cur, axis=1, keepdims=True)
        s_top = s_top + mx
        hit = jnp.where(cur == mx, iota, T)
        first = jnp.min(hit, axis=1, keepdims=True)
        cur = jnp.where(iota == first, jnp.float32(-jnp.inf), cur)

    c_bk = 0.5 * (s_top + s_soft)

    wu = jax.lax.dot_general(
        u[kk:kk + 1], clsw_ref[0], (((1,), (1,)), ((), ())),
        preferred_element_type=jnp.float32,
    )
    contrib = alpha[0, kk] * (c_bk * wu + clsb_ref[kk][None, :])
    if kk == 0:
        out_ref[...] = contrib
    else:
        out_ref[...] += contrib


def _finish_kernel(c0_ref, u_ref, clsw_ref, clsb_ref, beta_ref, alpha_ref,
                   out_ref, coeff_ref, *, B, T, top_m):
    k = pl.program_id(0)
    for kk in range(_K):
        @pl.when(k == kk)
        def _():
            _finish_step(kk, c0_ref, u_ref, clsw_ref, clsb_ref, beta_ref,
                         alpha_ref, out_ref, coeff_ref, B=B, T=T, top_m=top_m)


def kernel(E, v, m_logits, cls_W, cls_b, beta, alpha):
    B, T, D = E.shape
    K = v.shape[0]
    top_m = min(_TOP_M, T)

    mk = jax.nn.sigmoid(m_logits)
    vk = v * mk
    U = vk / (jnp.linalg.norm(vk, axis=1, keepdims=True) + _EPS)

    Tb = 2048
    c0 = pl.pallas_call(
        _matmul_kernel,
        grid=(B, T // Tb),
        in_specs=[
            pl.BlockSpec((1, Tb, D), lambda b, t: (b, t, 0)),
            pl.BlockSpec((K, D), lambda b, t: (0, 0)),
        ],
        out_specs=pl.BlockSpec((1, K, Tb), lambda b, t: (b, 0, t)),
        out_shape=jax.ShapeDtypeStruct((B, K, T), jnp.float32),
    )(E, U)

    c0_kbt = c0.transpose(1, 0, 2)

    finish = functools.partial(_finish_kernel, B=B, T=T, top_m=top_m)
    logits = pl.pallas_call(
        finish,
        grid=(K,),
        in_specs=[
            pl.BlockSpec((K, B, T), lambda k: (0, 0, 0)),
            pl.BlockSpec((K, D), lambda k: (0, 0)),
            pl.BlockSpec((1, _NUM_CLASSES, D), lambda k: (k, 0, 0)),
            pl.BlockSpec((K, _NUM_CLASSES), lambda k: (0, 0)),
            pl.BlockSpec((1, K), lambda k: (0, 0)),
            pl.BlockSpec((1, K), lambda k: (0, 0)),
        ],
        out_specs=pl.BlockSpec((B, _NUM_CLASSES), lambda k: (0, 0)),
        out_shape=jax.ShapeDtypeStruct((B, _NUM_CLASSES), jnp.float32),
        scratch_shapes=[pltpu.VMEM((K - 1, B, T), jnp.float32)],
    )(c0_kbt, U, cls_W, cls_b, beta.reshape(1, K), alpha.reshape(1, K))
    return logits
